# Initial kernel scaffold; baseline (speedup 1.0000x reference)
import jax
import jax.numpy as jnp
from jax.experimental import pallas as pl


def kernel(X, edge_index, edge_weight, adj_label, eps, W1, W_mean, W_logsig):
    raise NotImplementedError("write your pallas kernel here")


# speedup vs baseline: 5.3975x; 5.3975x over previous
import functools

import jax
import jax.numpy as jnp
from jax import lax
from jax.experimental import pallas as pl
from jax.experimental.pallas import tpu as pltpu
from jax.experimental.pallas import tpu_sc as plsc

_N = 4096
_E = 131072
_NX = 512
_NH = 256
_NZ = 64
_POS_WEIGHT = float(_N * _N - _E) / _E
_NORM_LOSS = (_N * _N) / float((_N * _N - _E) * 2)

_NC = 2
_NS = 16
_NW = _NC * _NS
_C = 128
_D = 128
_EPW = _E // _NW
_T = _EPW // _C
_RPS = _N // _NS

_HIGH = jax.lax.Precision.HIGHEST


def _make_spmm(nb):
    mesh = plsc.VectorSubcoreMesh(core_axis_name="c", subcore_axis_name="s")

    @functools.partial(
        pl.kernel,
        mesh=mesh,
        out_type=jax.ShapeDtypeStruct((nb, _NC, _N, _D), jnp.float32),
        scratch_types=[
            pltpu.VMEM((_EPW,), jnp.int32),
            pltpu.VMEM((_C,), jnp.int32),
            pltpu.VMEM((_EPW,), jnp.float32),
            pltpu.VMEM((_C * 16,), jnp.float32),
            pltpu.VMEM((_C, _D), jnp.float32),
        ] + [
            pltpu.VMEM_SHARED((_N, _D), jnp.float32) for _ in range(nb)
        ] + [
            pltpu.SemaphoreType.DMA,
        ],
    )
    def spmm(*refs):
        (h_hbm, src_hbm, dst_hbm, w_hbm, out_hbm,
         src_v, dst_v, w_v, wbc_v, rows_v) = refs[:10]
        accs = refs[10:10 + nb]
        sem = refs[10 + nb]

        c = lax.axis_index("c")
        s = lax.axis_index("s")
        wid = c * _NS + s
        eoff = wid * _EPW

        pltpu.sync_copy(src_hbm.at[pl.ds(eoff, _EPW)], src_v)
        pltpu.sync_copy(w_hbm.at[pl.ds(eoff, _EPW)], w_v)

        zv = jnp.zeros((16,), jnp.float32)

        def zero_body(i, carry):
            for j in range(_D // 16):
                rows_v[i, pl.ds(j * 16, 16)] = zv
            return carry

        lax.fori_loop(0, _C, zero_body, 0)
        for acc in accs:
            for b in range(_RPS // _C):
                pltpu.sync_copy(rows_v, acc.at[pl.ds(s * _RPS + b * _C, _C)])
        plsc.subcore_barrier()

        def do_chunk(t, blk):
            pltpu.async_copy(
                h_hbm.at[blk].at[plsc.Indices(src_v.at[pl.ds(t * _C, _C)])],
                rows_v, sem).wait()

            def mul_body(e, c2):
                wv16 = wbc_v[pl.ds(e * 16, 16)]
                for j in range(_D // 16):
                    rows_v[e, pl.ds(j * 16, 16)] = (
                        rows_v[e, pl.ds(j * 16, 16)] * wv16)
                return c2

            lax.fori_loop(0, _C, mul_body, 0)

            pltpu.sync_copy(rows_v, accs[blk].at[plsc.Indices(dst_v)],
                            add=True)

        for t in range(_T):
            pltpu.sync_copy(dst_hbm.at[pl.ds(eoff + t * _C, _C)], dst_v)

            def bc_body(g, carry, t=t):
                wv = w_v[pl.ds(t * _C + g * 16, 16)]
                for l in range(16):
                    wbc_v[pl.ds((g * 16 + l) * 16, 16)] = jnp.broadcast_to(
                        wv[l], (16,))
                return carry

            lax.fori_loop(0, _C // 16, bc_body, 0)
            for blk in range(nb):
                do_chunk(t, blk)
        plsc.subcore_barrier()

        for blk in range(nb):
            for b in range(_RPS // _C):
                r0 = s * _RPS + b * _C
                pltpu.sync_copy(accs[blk].at[pl.ds(r0, _C)], rows_v)
                pltpu.sync_copy(rows_v, out_hbm.at[blk, c, pl.ds(r0, _C)])

    return spmm


_spmm_h = _make_spmm(_NH // _D)
_spmm_z = _make_spmm(2 * _NZ // _D)


def _mm_kernel(x_ref, w_ref, o_ref):
    a = jnp.dot(x_ref[...], w_ref[...],
                precision=_HIGH, preferred_element_type=jnp.float32)
    o_ref[0] = a[:, :_D]
    o_ref[1] = a[:, _D:]


def _mid_kernel(s1_ref, w_ref, o_ref):
    h1a = jnp.maximum(s1_ref[0, 0] + s1_ref[0, 1], 0.0)
    h1b = jnp.maximum(s1_ref[1, 0] + s1_ref[1, 1], 0.0)
    o_ref[0] = (
        jnp.dot(h1a, w_ref[:_D], precision=_HIGH,
                preferred_element_type=jnp.float32)
        + jnp.dot(h1b, w_ref[_D:], precision=_HIGH,
                  preferred_element_type=jnp.float32))


def _z_kernel(s2_ref, eps_ref, z_ref, lat_ref):
    s2 = s2_ref[0, 0] + s2_ref[0, 1]
    zm = s2[:, :_NZ]
    zl = s2[:, _NZ:]
    sig = jnp.exp(zl)
    z_ref[...] = zm + eps_ref[...] * sig
    lat_ref[...] = jnp.sum(
        1.0 + 2.0 * zl - zm * zm - sig * sig).reshape(1, 1)


def _dec_kernel(zb_ref, zf_ref, lab_ref, a_ref, sum_ref):
    i = pl.program_id(0)
    a = lax.dot_general(zb_ref[...], zf_ref[...], (((1,), (1,)), ((), ())),
                        precision=_HIGH, preferred_element_type=jnp.float32)
    a_ref[...] = a
    lab = lab_ref[...]
    log_weight = 1.0 + (_POS_WEIGHT - 1.0) * lab
    ce = (1.0 - lab) * a + log_weight * (
        jnp.log1p(jnp.exp(-jnp.abs(a))) + jnp.maximum(-a, 0.0))
    part = jnp.sum(ce)

    @pl.when(i == 0)
    def _init():
        sum_ref[0, 0] = 0.0

    sum_ref[0, 0] += part


def kernel(X, edge_index, edge_weight, adj_label, eps, W1, W_mean, W_logsig):
    src = edge_index[0]
    dst = edge_index[1]
    wcat = jnp.concatenate([W_mean, W_logsig], axis=1)

    xw = pl.pallas_call(
        _mm_kernel,
        out_shape=jax.ShapeDtypeStruct((2, _N, _D), jnp.float32),
    )(X, W1)

    s1 = _spmm_h(xw, src, dst, edge_weight)

    h2 = pl.pallas_call(
        _mid_kernel,
        out_shape=jax.ShapeDtypeStruct((1, _N, _D), jnp.float32),
    )(s1, wcat)

    s2 = _spmm_z(h2, src, dst, edge_weight)

    z, lat = pl.pallas_call(
        _z_kernel,
        out_shape=(
            jax.ShapeDtypeStruct((_N, _NZ), jnp.float32),
            jax.ShapeDtypeStruct((1, 1), jnp.float32),
        ),
    )(s2, eps)

    blk = 256
    nblk = _N // blk
    a, ce_sum = pl.pallas_call(
        _dec_kernel,
        grid=(nblk,),
        in_specs=[
            pl.BlockSpec((blk, _NZ), lambda i: (i, 0)),
            pl.BlockSpec((_N, _NZ), lambda i: (0, 0)),
            pl.BlockSpec((blk, _N), lambda i: (i, 0)),
        ],
        out_specs=(
            pl.BlockSpec((blk, _N), lambda i: (i, 0)),
            pl.BlockSpec(memory_space=pltpu.SMEM, block_shape=(1, 1),
                         index_map=lambda i: (0, 0)),
        ),
        out_shape=(
            jax.ShapeDtypeStruct((_N, _N), jnp.float32),
            jax.ShapeDtypeStruct((1, 1), jnp.float32),
        ),
    )(z, z, adj_label)

    loss_latent = (-0.5 / (_N * _N)) * lat[0, 0]
    loss = _NORM_LOSS * ce_sum[0, 0] / (_N * _N) + loss_latent
    return (a, loss)


# speedup vs baseline: 6.9791x; 1.2930x over previous
import functools

import jax
import jax.numpy as jnp
from jax import lax
from jax.experimental import pallas as pl
from jax.experimental.pallas import tpu as pltpu
from jax.experimental.pallas import tpu_sc as plsc

_N = 4096
_E = 131072
_NX = 512
_NH = 256
_NZ = 64
_POS_WEIGHT = float(_N * _N - _E) / _E
_NORM_LOSS = (_N * _N) / float((_N * _N - _E) * 2)

_NC = 2
_NS = 16
_NW = _NC * _NS
_C = 128
_D = 128
_EPW = _E // _NW
_T = _EPW // _C
_RPS = _N // _NS

_HIGH = jax.lax.Precision.HIGHEST


def _make_spmm(nb):
    mesh = plsc.VectorSubcoreMesh(core_axis_name="c", subcore_axis_name="s")

    nring = 3

    @functools.partial(
        pl.kernel,
        mesh=mesh,
        out_type=jax.ShapeDtypeStruct((nb, _NC, _N, _D), jnp.float32),
        scratch_types=[
            pltpu.VMEM((_C,), jnp.float32),
        ] + [
            pltpu.VMEM((_C,), jnp.int32) for _ in range(nring)
        ] + [
            pltpu.VMEM((_C,), jnp.int32) for _ in range(nring)
        ] + [
            pltpu.VMEM((_C * 16,), jnp.float32) for _ in range(nring)
        ] + [
            pltpu.VMEM((_C, _D), jnp.float32) for _ in range(nring)
        ] + [
            pltpu.VMEM_SHARED((_N, _D), jnp.float32) for _ in range(nb)
        ] + [
            pltpu.SemaphoreType.DMA,
            pltpu.SemaphoreType.DMA,
        ],
    )
    def spmm(*refs):
        (h_hbm, src_hbm, dst_hbm, w_hbm, out_hbm, ws_c) = refs[:6]
        pos = 6
        srcs = refs[pos:pos + nring]; pos += nring
        dsts = refs[pos:pos + nring]; pos += nring
        wbcs = refs[pos:pos + nring]; pos += nring
        bufs = refs[pos:pos + nring]; pos += nring
        accs = refs[pos:pos + nb]; pos += nb
        sem_g, sem_s = refs[pos:pos + 2]

        c = lax.axis_index("c")
        s = lax.axis_index("s")
        wid = c * _NS + s
        eoff = wid * _EPW

        def stage_chunk(t):
            pltpu.sync_copy(dst_hbm.at[pl.ds(eoff + t * _C, _C)],
                            dsts[t % nring])
            pltpu.sync_copy(w_hbm.at[pl.ds(eoff + t * _C, _C)], ws_c)
            wbc_v = wbcs[t % nring]

            def bc_body(g, carry):
                wv = ws_c[pl.ds(g * 16, 16)]
                for l in range(16):
                    wbc_v[pl.ds((g * 16 + l) * 16, 16)] = jnp.broadcast_to(
                        wv[l], (16,))
                return carry

            lax.fori_loop(0, _C // 16, bc_body, 0)

        def stage_src(t):
            pltpu.sync_copy(src_hbm.at[pl.ds(eoff + t * _C, _C)],
                            srcs[t % nring])

        def gather(k):
            t, blk = divmod(k, nb)
            return pltpu.async_copy(
                h_hbm.at[blk].at[plsc.Indices(srcs[t % nring])],
                bufs[k % nring], sem_g)

        zv = jnp.zeros((16,), jnp.float32)

        def zero_body(i, carry):
            for j in range(_D // 16):
                bufs[0][i, pl.ds(j * 16, 16)] = zv
            return carry

        lax.fori_loop(0, _C, zero_body, 0)
        for acc in accs:
            for b in range(_RPS // _C):
                pltpu.sync_copy(bufs[0], acc.at[pl.ds(s * _RPS + b * _C, _C)])
        plsc.subcore_barrier()

        K = _T * nb
        sh = [None] * K
        stage_src(0)
        stage_chunk(0)
        gh = gather(0)
        for k in range(K):
            t, blk = divmod(k, nb)
            if k + 1 < K:
                tn, blkn = divmod(k + 1, nb)
                if k >= 2:
                    sh[k - 2].wait()
                if tn != t:
                    stage_src(tn)
                gh_next = gather(k + 1)
                if tn != t:
                    stage_chunk(tn)
            gh.wait()
            if k + 1 < K:
                gh = gh_next

            buf = bufs[k % nring]
            wbc_v = wbcs[t % nring]

            def mul_body(e, c2):
                wv16 = wbc_v[pl.ds(e * 16, 16)]
                for j in range(_D // 16):
                    buf[e, pl.ds(j * 16, 16)] = (
                        buf[e, pl.ds(j * 16, 16)] * wv16)
                return c2

            lax.fori_loop(0, _C, mul_body, 0)

            sh[k] = pltpu.async_copy(
                buf, accs[blk].at[plsc.Indices(dsts[t % nring])],
                sem_s, add=True)
        sh[K - 2].wait()
        sh[K - 1].wait()
        plsc.subcore_barrier()

        for blk in range(nb):
            for b in range(_RPS // _C):
                r0 = s * _RPS + b * _C
                pltpu.sync_copy(accs[blk].at[pl.ds(r0, _C)], bufs[0])
                pltpu.sync_copy(bufs[0], out_hbm.at[blk, c, pl.ds(r0, _C)])

    return spmm


_spmm_h = _make_spmm(_NH // _D)
_spmm_z = _make_spmm(2 * _NZ // _D)


def _mm_kernel(x_ref, w_ref, o_ref):
    a = jnp.dot(x_ref[...], w_ref[...],
                precision=_HIGH, preferred_element_type=jnp.float32)
    o_ref[0] = a[:, :_D]
    o_ref[1] = a[:, _D:]


def _mid_kernel(s1_ref, w_ref, o_ref):
    h1a = jnp.maximum(s1_ref[0, 0] + s1_ref[0, 1], 0.0)
    h1b = jnp.maximum(s1_ref[1, 0] + s1_ref[1, 1], 0.0)
    o_ref[0] = (
        jnp.dot(h1a, w_ref[:_D], precision=_HIGH,
                preferred_element_type=jnp.float32)
        + jnp.dot(h1b, w_ref[_D:], precision=_HIGH,
                  preferred_element_type=jnp.float32))


def _z_kernel(s2_ref, eps_ref, z_ref, lat_ref):
    s2 = s2_ref[0, 0] + s2_ref[0, 1]
    zm = s2[:, :_NZ]
    zl = s2[:, _NZ:]
    sig = jnp.exp(zl)
    z_ref[...] = zm + eps_ref[...] * sig
    lat_ref[...] = jnp.sum(
        1.0 + 2.0 * zl - zm * zm - sig * sig).reshape(1, 1)


def _dec_kernel(zb_ref, zf_ref, lab_ref, a_ref, sum_ref):
    i = pl.program_id(0)
    a = lax.dot_general(zb_ref[...], zf_ref[...], (((1,), (1,)), ((), ())),
                        precision=_HIGH, preferred_element_type=jnp.float32)
    a_ref[...] = a
    lab = lab_ref[...]
    log_weight = 1.0 + (_POS_WEIGHT - 1.0) * lab
    ce = (1.0 - lab) * a + log_weight * (
        jnp.log1p(jnp.exp(-jnp.abs(a))) + jnp.maximum(-a, 0.0))
    part = jnp.sum(ce)

    @pl.when(i == 0)
    def _init():
        sum_ref[0, 0] = 0.0

    sum_ref[0, 0] += part


def kernel(X, edge_index, edge_weight, adj_label, eps, W1, W_mean, W_logsig):
    src = edge_index[0]
    dst = edge_index[1]
    wcat = jnp.concatenate([W_mean, W_logsig], axis=1)

    xw = pl.pallas_call(
        _mm_kernel,
        out_shape=jax.ShapeDtypeStruct((2, _N, _D), jnp.float32),
    )(X, W1)

    s1 = _spmm_h(xw, src, dst, edge_weight)

    h2 = pl.pallas_call(
        _mid_kernel,
        out_shape=jax.ShapeDtypeStruct((1, _N, _D), jnp.float32),
    )(s1, wcat)

    s2 = _spmm_z(h2, src, dst, edge_weight)

    z, lat = pl.pallas_call(
        _z_kernel,
        out_shape=(
            jax.ShapeDtypeStruct((_N, _NZ), jnp.float32),
            jax.ShapeDtypeStruct((1, 1), jnp.float32),
        ),
    )(s2, eps)

    blk = 256
    nblk = _N // blk
    a, ce_sum = pl.pallas_call(
        _dec_kernel,
        grid=(nblk,),
        in_specs=[
            pl.BlockSpec((blk, _NZ), lambda i: (i, 0)),
            pl.BlockSpec((_N, _NZ), lambda i: (0, 0)),
            pl.BlockSpec((blk, _N), lambda i: (i, 0)),
        ],
        out_specs=(
            pl.BlockSpec((blk, _N), lambda i: (i, 0)),
            pl.BlockSpec(memory_space=pltpu.SMEM, block_shape=(1, 1),
                         index_map=lambda i: (0, 0)),
        ),
        out_shape=(
            jax.ShapeDtypeStruct((_N, _N), jnp.float32),
            jax.ShapeDtypeStruct((1, 1), jnp.float32),
        ),
    )(z, z, adj_label)

    loss_latent = (-0.5 / (_N * _N)) * lat[0, 0]
    loss = _NORM_LOSS * ce_sum[0, 0] / (_N * _N) + loss_latent
    return (a, loss)


# speedup vs baseline: 7.9398x; 1.1376x over previous
import functools

import jax
import jax.numpy as jnp
from jax import lax
from jax.experimental import pallas as pl
from jax.experimental.pallas import tpu as pltpu
from jax.experimental.pallas import tpu_sc as plsc

_N = 4096
_E = 131072
_NX = 512
_NH = 256
_NZ = 64
_POS_WEIGHT = float(_N * _N - _E) / _E
_NORM_LOSS = (_N * _N) / float((_N * _N - _E) * 2)

_NC = 2
_NS = 16
_NW = _NC * _NS
_C = 128
_D = 128
_EPW = _E // _NW
_T = _EPW // _C
_RPS = _N // _NS

_HIGH = jax.lax.Precision.HIGHEST


def _make_spmm(nb):
    mesh = plsc.VectorSubcoreMesh(core_axis_name="c", subcore_axis_name="s")

    nring = 3

    @functools.partial(
        pl.kernel,
        mesh=mesh,
        out_type=jax.ShapeDtypeStruct((nb, _NC, _N, _D), jnp.float32),
        scratch_types=[
            pltpu.VMEM((_C,), jnp.float32),
        ] + [
            pltpu.VMEM((_C,), jnp.int32) for _ in range(nring)
        ] + [
            pltpu.VMEM((_C,), jnp.int32) for _ in range(nring)
        ] + [
            pltpu.VMEM((_C * 16,), jnp.float32) for _ in range(nring)
        ] + [
            pltpu.VMEM((_C, _D), jnp.float32) for _ in range(nring)
        ] + [
            pltpu.VMEM_SHARED((_N, _D), jnp.float32) for _ in range(nb)
        ] + [
            pltpu.SemaphoreType.DMA,
            pltpu.SemaphoreType.DMA,
        ],
    )
    def spmm(*refs):
        (h_hbm, src_hbm, dst_hbm, w_hbm, out_hbm, ws_c) = refs[:6]
        pos = 6
        srcs = refs[pos:pos + nring]; pos += nring
        dsts = refs[pos:pos + nring]; pos += nring
        wbcs = refs[pos:pos + nring]; pos += nring
        bufs = refs[pos:pos + nring]; pos += nring
        accs = refs[pos:pos + nb]; pos += nb
        sem_g, sem_s = refs[pos:pos + 2]

        c = lax.axis_index("c")
        s = lax.axis_index("s")
        wid = c * _NS + s
        eoff = wid * _EPW

        def stage_chunk(t):
            pltpu.sync_copy(dst_hbm.at[pl.ds(eoff + t * _C, _C)],
                            dsts[t % nring])
            pltpu.sync_copy(w_hbm.at[pl.ds(eoff + t * _C, _C)], ws_c)
            wbc_v = wbcs[t % nring]

            @plsc.parallel_loop(0, _C // 16, 1)
            def bc_body(g):
                wv = ws_c[pl.ds(g * 16, 16)]
                for l in range(16):
                    wbc_v[pl.ds((g * 16 + l) * 16, 16)] = jnp.broadcast_to(
                        wv[l], (16,))

        def stage_src(t):
            pltpu.sync_copy(src_hbm.at[pl.ds(eoff + t * _C, _C)],
                            srcs[t % nring])

        def gather(k):
            t, blk = divmod(k, nb)
            return pltpu.async_copy(
                h_hbm.at[blk].at[plsc.Indices(srcs[t % nring])],
                bufs[k % nring], sem_g)

        zv = jnp.zeros((16,), jnp.float32)

        def zero_body(i, carry):
            for j in range(_D // 16):
                bufs[0][i, pl.ds(j * 16, 16)] = zv
            return carry

        lax.fori_loop(0, _C, zero_body, 0)
        for acc in accs:
            for b in range(_RPS // _C):
                pltpu.sync_copy(bufs[0], acc.at[pl.ds(s * _RPS + b * _C, _C)])
        plsc.subcore_barrier()

        K = _T * nb
        sh = [None] * K
        stage_src(0)
        stage_chunk(0)
        gh = gather(0)
        for k in range(K):
            t, blk = divmod(k, nb)
            if k + 1 < K:
                tn, blkn = divmod(k + 1, nb)
                if k >= 2:
                    sh[k - 2].wait()
                if tn != t:
                    stage_src(tn)
                gh_next = gather(k + 1)
                if tn != t:
                    stage_chunk(tn)
            gh.wait()
            if k + 1 < K:
                gh = gh_next

            buf = bufs[k % nring]
            wbc_v = wbcs[t % nring]

            @plsc.parallel_loop(0, _C, 1)
            def mul_body(e):
                wv16 = wbc_v[pl.ds(e * 16, 16)]
                for j in range(_D // 16):
                    buf[e, pl.ds(j * 16, 16)] = (
                        buf[e, pl.ds(j * 16, 16)] * wv16)

            sh[k] = pltpu.async_copy(
                buf, accs[blk].at[plsc.Indices(dsts[t % nring])],
                sem_s, add=True)
        sh[K - 2].wait()
        sh[K - 1].wait()
        plsc.subcore_barrier()

        for blk in range(nb):
            for b in range(_RPS // _C):
                r0 = s * _RPS + b * _C
                pltpu.sync_copy(accs[blk].at[pl.ds(r0, _C)], bufs[0])
                pltpu.sync_copy(bufs[0], out_hbm.at[blk, c, pl.ds(r0, _C)])

    return spmm


_spmm_h = _make_spmm(_NH // _D)
_spmm_z = _make_spmm(2 * _NZ // _D)


def _mm_kernel(x_ref, w_ref, o_ref):
    a = jnp.dot(x_ref[...], w_ref[...],
                precision=_HIGH, preferred_element_type=jnp.float32)
    o_ref[0] = a[:, :_D]
    o_ref[1] = a[:, _D:]


def _mid_kernel(s1_ref, w_ref, o_ref):
    h1a = jnp.maximum(s1_ref[0, 0] + s1_ref[0, 1], 0.0)
    h1b = jnp.maximum(s1_ref[1, 0] + s1_ref[1, 1], 0.0)
    o_ref[0] = (
        jnp.dot(h1a, w_ref[:_D], precision=_HIGH,
                preferred_element_type=jnp.float32)
        + jnp.dot(h1b, w_ref[_D:], precision=_HIGH,
                  preferred_element_type=jnp.float32))


def _z_kernel(s2_ref, eps_ref, z_ref, lat_ref):
    s2 = s2_ref[0, 0] + s2_ref[0, 1]
    zm = s2[:, :_NZ]
    zl = s2[:, _NZ:]
    sig = jnp.exp(zl)
    z_ref[...] = zm + eps_ref[...] * sig
    lat_ref[...] = jnp.sum(
        1.0 + 2.0 * zl - zm * zm - sig * sig).reshape(1, 1)


def _dec_kernel(zb_ref, zf_ref, lab_ref, a_ref, sum_ref):
    i = pl.program_id(0)
    a = lax.dot_general(zb_ref[...], zf_ref[...], (((1,), (1,)), ((), ())),
                        precision=_HIGH, preferred_element_type=jnp.float32)
    a_ref[...] = a
    lab = lab_ref[...]
    log_weight = 1.0 + (_POS_WEIGHT - 1.0) * lab
    ce = (1.0 - lab) * a + log_weight * (
        jnp.log1p(jnp.exp(-jnp.abs(a))) + jnp.maximum(-a, 0.0))
    part = jnp.sum(ce)

    @pl.when(i == 0)
    def _init():
        sum_ref[0, 0] = 0.0

    sum_ref[0, 0] += part


def kernel(X, edge_index, edge_weight, adj_label, eps, W1, W_mean, W_logsig):
    src = edge_index[0]
    dst = edge_index[1]
    wcat = jnp.concatenate([W_mean, W_logsig], axis=1)

    xw = pl.pallas_call(
        _mm_kernel,
        out_shape=jax.ShapeDtypeStruct((2, _N, _D), jnp.float32),
    )(X, W1)

    s1 = _spmm_h(xw, src, dst, edge_weight)

    h2 = pl.pallas_call(
        _mid_kernel,
        out_shape=jax.ShapeDtypeStruct((1, _N, _D), jnp.float32),
    )(s1, wcat)

    s2 = _spmm_z(h2, src, dst, edge_weight)

    z, lat = pl.pallas_call(
        _z_kernel,
        out_shape=(
            jax.ShapeDtypeStruct((_N, _NZ), jnp.float32),
            jax.ShapeDtypeStruct((1, 1), jnp.float32),
        ),
    )(s2, eps)

    blk = 256
    nblk = _N // blk
    a, ce_sum = pl.pallas_call(
        _dec_kernel,
        grid=(nblk,),
        in_specs=[
            pl.BlockSpec((blk, _NZ), lambda i: (i, 0)),
            pl.BlockSpec((_N, _NZ), lambda i: (0, 0)),
            pl.BlockSpec((blk, _N), lambda i: (i, 0)),
        ],
        out_specs=(
            pl.BlockSpec((blk, _N), lambda i: (i, 0)),
            pl.BlockSpec(memory_space=pltpu.SMEM, block_shape=(1, 1),
                         index_map=lambda i: (0, 0)),
        ),
        out_shape=(
            jax.ShapeDtypeStruct((_N, _N), jnp.float32),
            jax.ShapeDtypeStruct((1, 1), jnp.float32),
        ),
    )(z, z, adj_label)

    loss_latent = (-0.5 / (_N * _N)) * lat[0, 0]
    loss = _NORM_LOSS * ce_sum[0, 0] / (_N * _N) + loss_latent
    return (a, loss)


# speedup vs baseline: 8.9472x; 1.1269x over previous
import functools

import jax
import jax.numpy as jnp
from jax import lax
from jax.experimental import pallas as pl
from jax.experimental.pallas import tpu as pltpu
from jax.experimental.pallas import tpu_sc as plsc

_N = 4096
_E = 131072
_NX = 512
_NH = 256
_NZ = 64
_POS_WEIGHT = float(_N * _N - _E) / _E
_NORM_LOSS = (_N * _N) / float((_N * _N - _E) * 2)

_NC = 2
_NS = 16
_NW = _NC * _NS
_C = 128
_D = 128
_EPW = _E // _NW
_T = _EPW // _C
_RPS = _N // _NS

_HIGH = jax.lax.Precision.DEFAULT


def _make_spmm(nb):
    mesh = plsc.VectorSubcoreMesh(core_axis_name="c", subcore_axis_name="s")

    nring = 3

    @functools.partial(
        pl.kernel,
        mesh=mesh,
        out_type=jax.ShapeDtypeStruct((nb, _NC, _N, _D), jnp.float32),
        scratch_types=[
            pltpu.VMEM((_C,), jnp.float32),
        ] + [
            pltpu.VMEM((_C,), jnp.int32) for _ in range(nring)
        ] + [
            pltpu.VMEM((_C,), jnp.int32) for _ in range(nring)
        ] + [
            pltpu.VMEM((_C * 16,), jnp.float32) for _ in range(nring)
        ] + [
            pltpu.VMEM((_C, _D), jnp.float32) for _ in range(nring)
        ] + [
            pltpu.VMEM_SHARED((_N, _D), jnp.float32) for _ in range(nb)
        ] + [
            pltpu.SemaphoreType.DMA,
            pltpu.SemaphoreType.DMA,
        ],
    )
    def spmm(*refs):
        (h_hbm, src_hbm, dst_hbm, w_hbm, out_hbm, ws_c) = refs[:6]
        pos = 6
        srcs = refs[pos:pos + nring]; pos += nring
        dsts = refs[pos:pos + nring]; pos += nring
        wbcs = refs[pos:pos + nring]; pos += nring
        bufs = refs[pos:pos + nring]; pos += nring
        accs = refs[pos:pos + nb]; pos += nb
        sem_g, sem_s = refs[pos:pos + 2]

        c = lax.axis_index("c")
        s = lax.axis_index("s")
        wid = c * _NS + s
        eoff = wid * _EPW

        def stage_chunk(t):
            pltpu.sync_copy(dst_hbm.at[pl.ds(eoff + t * _C, _C)],
                            dsts[t % nring])
            pltpu.sync_copy(w_hbm.at[pl.ds(eoff + t * _C, _C)], ws_c)
            wbc_v = wbcs[t % nring]

            @plsc.parallel_loop(0, _C // 16, 1)
            def bc_body(g):
                wv = ws_c[pl.ds(g * 16, 16)]
                for l in range(16):
                    wbc_v[pl.ds((g * 16 + l) * 16, 16)] = jnp.broadcast_to(
                        wv[l], (16,))

        def stage_src(t):
            pltpu.sync_copy(src_hbm.at[pl.ds(eoff + t * _C, _C)],
                            srcs[t % nring])

        def gather(k):
            t, blk = divmod(k, nb)
            return pltpu.async_copy(
                h_hbm.at[blk].at[plsc.Indices(srcs[t % nring])],
                bufs[k % nring], sem_g)

        zv = jnp.zeros((16,), jnp.float32)

        def zero_body(i, carry):
            for j in range(_D // 16):
                bufs[0][i, pl.ds(j * 16, 16)] = zv
            return carry

        lax.fori_loop(0, _C, zero_body, 0)
        for acc in accs:
            for b in range(_RPS // _C):
                pltpu.sync_copy(bufs[0], acc.at[pl.ds(s * _RPS + b * _C, _C)])
        plsc.subcore_barrier()

        K = _T * nb
        sh = [None] * K
        stage_src(0)
        stage_chunk(0)
        gh = gather(0)
        for k in range(K):
            t, blk = divmod(k, nb)
            if k + 1 < K:
                tn, blkn = divmod(k + 1, nb)
                if k >= 2:
                    sh[k - 2].wait()
                if tn != t:
                    stage_src(tn)
                gh_next = gather(k + 1)
                if tn != t:
                    stage_chunk(tn)
            gh.wait()
            if k + 1 < K:
                gh = gh_next

            buf = bufs[k % nring]
            wbc_v = wbcs[t % nring]

            @plsc.parallel_loop(0, _C, 1, unroll=2)
            def mul_body(e):
                wv16 = wbc_v[pl.ds(e * 16, 16)]
                for j in range(_D // 16):
                    buf[e, pl.ds(j * 16, 16)] = (
                        buf[e, pl.ds(j * 16, 16)] * wv16)

            sh[k] = pltpu.async_copy(
                buf, accs[blk].at[plsc.Indices(dsts[t % nring])],
                sem_s, add=True)
        sh[K - 2].wait()
        sh[K - 1].wait()
        plsc.subcore_barrier()

        for blk in range(nb):
            for b in range(_RPS // _C):
                r0 = s * _RPS + b * _C
                pltpu.sync_copy(accs[blk].at[pl.ds(r0, _C)], bufs[0])
                pltpu.sync_copy(bufs[0], out_hbm.at[blk, c, pl.ds(r0, _C)])

    return spmm


_spmm_h = _make_spmm(_NH // _D)
_spmm_z = _make_spmm(2 * _NZ // _D)


def _mm_kernel(x_ref, w_ref, o_ref):
    a = jnp.dot(x_ref[...], w_ref[...],
                precision=_HIGH, preferred_element_type=jnp.float32)
    o_ref[0] = a[:, :_D]
    o_ref[1] = a[:, _D:]


def _mid_kernel(s1_ref, w_ref, o_ref):
    h1a = jnp.maximum(s1_ref[0, 0] + s1_ref[0, 1], 0.0)
    h1b = jnp.maximum(s1_ref[1, 0] + s1_ref[1, 1], 0.0)
    o_ref[0] = (
        jnp.dot(h1a, w_ref[:_D], precision=_HIGH,
                preferred_element_type=jnp.float32)
        + jnp.dot(h1b, w_ref[_D:], precision=_HIGH,
                  preferred_element_type=jnp.float32))


def _z_kernel(s2_ref, eps_ref, z_ref, lat_ref):
    s2 = s2_ref[0, 0] + s2_ref[0, 1]
    zm = s2[:, :_NZ]
    zl = s2[:, _NZ:]
    sig = jnp.exp(zl)
    z_ref[...] = zm + eps_ref[...] * sig
    lat_ref[...] = jnp.sum(
        1.0 + 2.0 * zl - zm * zm - sig * sig).reshape(1, 1)


def _dec_kernel(zb_ref, zf_ref, lab_ref, a_ref, sum_ref):
    i = pl.program_id(0)
    a = lax.dot_general(zb_ref[...], zf_ref[...], (((1,), (1,)), ((), ())),
                        precision=_HIGH, preferred_element_type=jnp.float32)
    a_ref[...] = a
    lab = lab_ref[...]
    log_weight = 1.0 + (_POS_WEIGHT - 1.0) * lab
    ce = (1.0 - lab) * a + log_weight * (
        jnp.log1p(jnp.exp(-jnp.abs(a))) + jnp.maximum(-a, 0.0))
    part = jnp.sum(ce)

    @pl.when(i == 0)
    def _init():
        sum_ref[0, 0] = 0.0

    sum_ref[0, 0] += part


def kernel(X, edge_index, edge_weight, adj_label, eps, W1, W_mean, W_logsig):
    src = edge_index[0]
    dst = edge_index[1]
    wcat = jnp.concatenate([W_mean, W_logsig], axis=1)

    xw = pl.pallas_call(
        _mm_kernel,
        out_shape=jax.ShapeDtypeStruct((2, _N, _D), jnp.float32),
    )(X, W1)

    s1 = _spmm_h(xw, src, dst, edge_weight)

    h2 = pl.pallas_call(
        _mid_kernel,
        out_shape=jax.ShapeDtypeStruct((1, _N, _D), jnp.float32),
    )(s1, wcat)

    s2 = _spmm_z(h2, src, dst, edge_weight)

    z, lat = pl.pallas_call(
        _z_kernel,
        out_shape=(
            jax.ShapeDtypeStruct((_N, _NZ), jnp.float32),
            jax.ShapeDtypeStruct((1, 1), jnp.float32),
        ),
    )(s2, eps)

    blk = 256
    nblk = _N // blk
    a, ce_sum = pl.pallas_call(
        _dec_kernel,
        grid=(nblk,),
        in_specs=[
            pl.BlockSpec((blk, _NZ), lambda i: (i, 0)),
            pl.BlockSpec((_N, _NZ), lambda i: (0, 0)),
            pl.BlockSpec((blk, _N), lambda i: (i, 0)),
        ],
        out_specs=(
            pl.BlockSpec((blk, _N), lambda i: (i, 0)),
            pl.BlockSpec(memory_space=pltpu.SMEM, block_shape=(1, 1),
                         index_map=lambda i: (0, 0)),
        ),
        out_shape=(
            jax.ShapeDtypeStruct((_N, _N), jnp.float32),
            jax.ShapeDtypeStruct((1, 1), jnp.float32),
        ),
    )(z, z, adj_label)

    loss_latent = (-0.5 / (_N * _N)) * lat[0, 0]
    loss = _NORM_LOSS * ce_sum[0, 0] / (_N * _N) + loss_latent
    return (a, loss)
